# baseline (speedup 1.0000x reference)
import functools
from typing import Any

import jax
import jax.numpy as jnp
from jax import lax
from jax.experimental import pallas as pl
from jax.experimental.pallas import tpu as pltpu
from jax.experimental.pallas import tpu_sc as plsc

_NCORES = 2
_NSUB = 16
_NW = _NCORES * _NSUB
_NLANES = 16
_CHUNK = 80
_NSTREAM = 5


def _sc_counts(dst, N):
    E = dst.shape[0]
    D = 128
    NP = ((N + 127) // 128) * 128
    RPS = NP // _NSUB
    EPW = E // _NW
    n_chunks = EPW // _CHUNK
    assert EPW % _CHUNK == 0

    mesh = plsc.VectorSubcoreMesh(core_axis_name="c", subcore_axis_name="s")
    out_type = [jax.ShapeDtypeStruct((_NCORES, NP, D), jnp.float32)]
    scratch = [
        pltpu.VMEM((_CHUNK,), jnp.int32),
        pltpu.VMEM((_CHUNK, D), jnp.float32),
        pltpu.VMEM_SHARED((NP, D), jnp.float32),
    ]

    def body(dst_hbm, c_out, dst_v, ones_v, acc):
        cid = lax.axis_index("c")
        sid = lax.axis_index("s")
        wid = sid * _NCORES + cid
        rsl = pl.ds(sid * RPS, RPS)
        zrow = jnp.zeros((_NLANES,), jnp.float32)

        def zbuf(e, _):
            for j in range(D // _NLANES):
                ones_v[e, pl.ds(j * _NLANES, _NLANES)] = zrow
            return 0
        lax.fori_loop(0, _CHUNK, zbuf, 0)
        nfull, rem = RPS // _CHUNK, RPS % _CHUNK
        for k in range(nfull):
            pltpu.sync_copy(ones_v, acc.at[pl.ds(sid * RPS + k * _CHUNK, _CHUNK)])
        if rem:
            pltpu.sync_copy(ones_v.at[pl.ds(0, rem)],
                            acc.at[pl.ds(sid * RPS + nfull * _CHUNK, rem)])
        one_row = jnp.where(lax.iota(jnp.int32, _NLANES) == 0, 1.0, 0.0)

        def initones(e, _):
            ones_v[e, pl.ds(0, _NLANES)] = one_row
            return 0
        lax.fori_loop(0, _CHUNK, initones, 0)
        plsc.subcore_barrier()

        def chunk(g, _):
            base = wid * EPW + g * _CHUNK
            pltpu.sync_copy(dst_hbm.at[pl.ds(base, _CHUNK)], dst_v)
            pltpu.sync_copy(ones_v, acc.at[dst_v], add=True)
            return 0
        lax.fori_loop(0, n_chunks, chunk, 0)
        plsc.subcore_barrier()
        pltpu.sync_copy(acc.at[rsl], c_out.at[cid, rsl])

    return pl.kernel(body, out_type=out_type, mesh=mesh, scratch_types=scratch)(dst)


def _sc_segsum(h, src3d, dst3d, ew):
    N, D = h.shape
    E = ew.shape[0] // _NLANES
    NP = ((N + 127) // 128) * 128
    RPS = NP // _NSUB
    EPW = E // _NW
    NS = 1
    IB = NS * _CHUNK
    n_iter = EPW // IB
    assert EPW % IB == 0 and n_iter % 2 == 1 and NP % _NSUB == 0

    mesh = plsc.VectorSubcoreMesh(core_axis_name="c", subcore_axis_name="s")
    out_type = [jax.ShapeDtypeStruct((_NCORES, NP, D), jnp.float32)]
    scratch = [
        pltpu.VMEM((1, NS, _CHUNK), jnp.int32),
        pltpu.VMEM((1, NS, _CHUNK), jnp.int32),
        pltpu.VMEM((1, NS, _CHUNK), jnp.int32),
        pltpu.VMEM((1, NS, _CHUNK), jnp.int32),
        pltpu.VMEM((IB * _NLANES,), jnp.float32),
        pltpu.VMEM((IB * _NLANES,), jnp.float32),
        pltpu.VMEM((IB, D), jnp.float32),
        pltpu.VMEM((IB, D), jnp.float32),
        pltpu.VMEM_SHARED((NP, D), jnp.float32),
        pltpu.SemaphoreType.DMA, pltpu.SemaphoreType.DMA,
        pltpu.SemaphoreType.DMA, pltpu.SemaphoreType.DMA,
    ]

    def body(h_hbm, src3_hbm, dst3_hbm, ew_hbm, p_out,
             sb0, sb1, db0, db1, eb0, eb1, rb0, rb1, acc,
             gs0, gs1, ss0, ss1):
        srcb, dstb, ewb, rows = (sb0, sb1), (db0, db1), (eb0, eb1), (rb0, rb1)
        gsem, ssem = (gs0, gs1), (ss0, ss1)
        cid = lax.axis_index("c")
        sid = lax.axis_index("s")
        wid = sid * _NCORES + cid
        rsl = pl.ds(sid * RPS, RPS)
        zrow = jnp.zeros((_NLANES,), jnp.float32)

        def zbuf(e, _):
            for j in range(D // _NLANES):
                rb0[e, pl.ds(j * _NLANES, _NLANES)] = zrow
            return 0
        lax.fori_loop(0, IB, zbuf, 0)
        nfull, rem = RPS // IB, RPS % IB
        for k in range(nfull):
            pltpu.sync_copy(rb0, acc.at[pl.ds(sid * RPS + k * IB, IB)])
        if rem:
            pltpu.sync_copy(rb0.at[pl.ds(0, rem)],
                            acc.at[pl.ds(sid * RPS + nfull * IB, rem)])
        plsc.subcore_barrier()

        def load_idx(g, b):
            blk = wid * n_iter + g
            pltpu.sync_copy(src3_hbm.at[pl.ds(blk, 1)], srcb[b])
            pltpu.sync_copy(dst3_hbm.at[pl.ds(blk, 1)], dstb[b])
            pltpu.sync_copy(
                ew_hbm.at[pl.ds((wid * EPW + g * IB) * _NLANES, IB * _NLANES)],
                ewb[b])

        def start_gathers(b):
            for j in range(NS):
                pltpu.async_copy(h_hbm.at[srcb[b].at[0, j]],
                                 rows[b].at[pl.ds(j * _CHUNK, _CHUNK)], gsem[b])

        def drain_gathers(b):
            for j in range(NS):
                pltpu.make_async_copy(
                    h_hbm.at[pl.ds(0, _CHUNK)],
                    rows[b].at[pl.ds(j * _CHUNK, _CHUNK)], gsem[b]).wait()

        def start_scatters(b):
            for j in range(NS):
                pltpu.async_copy(rows[b].at[pl.ds(j * _CHUNK, _CHUNK)],
                                 acc.at[dstb[b].at[0, j]], ssem[b], add=True)

        def drain_scatters(b):
            for j in range(NS):
                pltpu.make_async_copy(
                    rows[b].at[pl.ds(j * _CHUNK, _CHUNK)],
                    acc.at[pl.ds(0, _CHUNK)], ssem[b]).wait()

        def scale(b):
            def s1(e, _):
                w = ewb[b][pl.ds(e * _NLANES, _NLANES)]
                for j in range(D // _NLANES):
                    sl = pl.ds(j * _NLANES, _NLANES)
                    rows[b][e, sl] = rows[b][e, sl] * w
                return 0
            lax.fori_loop(0, IB, s1, 0)

        load_idx(0, 0)
        start_gathers(0)

        @pl.loop(0, n_iter - 1, step=2)
        def _(i0):
            for b in (0, 1):
                g = i0 + b
                nb = 1 - b

                @pl.when(g >= 1)
                def _():
                    drain_scatters(nb)
                load_idx(g + 1, nb)
                start_gathers(nb)
                drain_gathers(b)
                scale(b)
                start_scatters(b)

        drain_gathers(0)
        scale(0)
        start_scatters(0)
        drain_scatters(0)
        drain_scatters(1)
        plsc.subcore_barrier()
        pltpu.sync_copy(acc.at[rsl], p_out.at[cid, rsl])

    return pl.kernel(body, out_type=out_type, mesh=mesh, scratch_types=scratch)(
        h, src3d, dst3d, ew)


def _sc_gather2(h, src3d, dst3d, E):
    N, D = h.shape
    EPW = E // _NW
    IB = _NSTREAM * _CHUNK
    n_iter = EPW // IB
    assert EPW % IB == 0 and n_iter % 2 == 1

    mesh = plsc.VectorSubcoreMesh(core_axis_name="c", subcore_axis_name="s")
    out_type = [jax.ShapeDtypeStruct((E, D), jnp.float32),
                jax.ShapeDtypeStruct((E, D), jnp.float32)]
    scratch = [
        pltpu.VMEM((1, _NSTREAM, _CHUNK), jnp.int32),
        pltpu.VMEM((1, _NSTREAM, _CHUNK), jnp.int32),
        pltpu.VMEM((IB, D), jnp.float32),
        pltpu.VMEM((IB, D), jnp.float32),
        pltpu.SemaphoreType.DMA, pltpu.SemaphoreType.DMA,
        pltpu.SemaphoreType.DMA, pltpu.SemaphoreType.DMA,
    ]

    def body(h_hbm, src3_hbm, dst3_hbm, hs_out, hd_out,
             ib0, ib1, rb0, rb1, gs0, gs1, ws0, ws1):
        idxb, rows = (ib0, ib1), (rb0, rb1)
        gsem, wsem = (gs0, gs1), (ws0, ws1)
        cid = lax.axis_index("c")
        sid = lax.axis_index("s")
        wid = sid * _NCORES + cid

        for idx3_hbm, out_hbm in ((src3_hbm, hs_out), (dst3_hbm, hd_out)):
            def load_idx(g, b):
                pltpu.sync_copy(idx3_hbm.at[pl.ds(wid * n_iter + g, 1)], idxb[b])

            def start_gathers(b):
                for j in range(_NSTREAM):
                    pltpu.async_copy(h_hbm.at[idxb[b].at[0, j]],
                                     rows[b].at[pl.ds(j * _CHUNK, _CHUNK)],
                                     gsem[b])

            def drain_gathers(b):
                for j in range(_NSTREAM):
                    pltpu.make_async_copy(
                        h_hbm.at[pl.ds(0, _CHUNK)],
                        rows[b].at[pl.ds(j * _CHUNK, _CHUNK)], gsem[b]).wait()

            def start_write(g, b):
                pltpu.async_copy(rows[b],
                                 out_hbm.at[pl.ds(wid * EPW + g * IB, IB)],
                                 wsem[b])

            def drain_write(g, b):
                pltpu.make_async_copy(
                    rows[b], out_hbm.at[pl.ds(wid * EPW + g * IB, IB)],
                    wsem[b]).wait()

            load_idx(0, 0)
            start_gathers(0)

            @pl.loop(0, n_iter - 1, step=2)
            def _(i0):
                for b in (0, 1):
                    g = i0 + b
                    nb = 1 - b

                    @pl.when(g >= 1)
                    def _():
                        drain_write(g - 1, nb)
                    load_idx(g + 1, nb)
                    start_gathers(nb)
                    drain_gathers(b)
                    start_write(g, b)

            drain_gathers(0)
            start_write(n_iter - 1, 0)
            drain_write(n_iter - 1, 0)
            drain_write(n_iter - 2, 1)

    return pl.kernel(body, out_type=out_type, mesh=mesh, scratch_types=scratch)(
        h, src3d, dst3d)


def _edge_head_body(hs_ref, hd_ref, r_ref,
                    wsrc_ref, wdst_ref, wr_ref, ball_ref,
                    w2c_ref, b2c_ref, w2p_ref, b2p_ref, w3c_ref, b3c_ref,
                    w3p_ref, b3p_ref, logit_ref, par_ref):
    hs = hs_ref[...]
    hd = hd_ref[...]
    r = r_ref[...]
    u = (jnp.dot(hs, wsrc_ref[...], preferred_element_type=jnp.float32)
         + jnp.dot(hd, wdst_ref[...], preferred_element_type=jnp.float32)
         + jnp.dot(r, wr_ref[...], preferred_element_type=jnp.float32)
         + ball_ref[...])
    u = jnp.maximum(u, 0.0)
    c = u[:, :128]
    q = u[:, 128:]
    c2 = jnp.maximum(jnp.dot(c, w2c_ref[...], preferred_element_type=jnp.float32)
                     + b2c_ref[...], 0.0)
    q2 = jnp.maximum(jnp.dot(q, w2p_ref[...], preferred_element_type=jnp.float32)
                     + b2p_ref[...], 0.0)
    lg = jnp.dot(c2, w3c_ref[...], preferred_element_type=jnp.float32) + b3c_ref[...]
    logit_ref[...] = lg
    pp = jnp.dot(q2, w3p_ref[...], preferred_element_type=jnp.float32) + b3p_ref[...]
    par_ref[...] = jax.nn.softplus(pp) + 1e-6


def _edge_head(hs, hd, r, wsrc, wdst, wr, ball, w2c, b2c,
               w2p, b2p, w3c, b3c, w3p, b3p, BE):
    E = hs.shape[0]
    grid = (E // BE,)
    wb = lambda shape: pl.BlockSpec(shape, lambda i: tuple(0 for _ in shape))
    return pl.pallas_call(
        _edge_head_body,
        grid=grid,
        in_specs=[
            pl.BlockSpec((BE, 128), lambda i: (i, 0)),
            pl.BlockSpec((BE, 128), lambda i: (i, 0)),
            pl.BlockSpec((BE, 32), lambda i: (i, 0)),
            wb((128, 256)), wb((128, 256)), wb((32, 256)), wb((1, 256)),
            wb((128, 64)), wb((1, 64)), wb((128, 64)), wb((1, 64)),
            wb((64, 1)), wb((1, 1)), wb((64, 4)), wb((1, 4)),
        ],
        out_specs=[
            pl.BlockSpec((BE, 1), lambda i: (i, 0)),
            pl.BlockSpec((BE, 4), lambda i: (i, 0)),
        ],
        out_shape=[
            jax.ShapeDtypeStruct((E, 1), jnp.float32),
            jax.ShapeDtypeStruct((E, 4), jnp.float32),
        ],
    )(hs, hd, r, wsrc, wdst, wr, ball,
      w2c, b2c, w2p, b2p, w3c, b3c, w3p, b3p)


def _layer_norm(x, g, b):
    m = x.mean(-1, keepdims=True)
    v = ((x - m) ** 2).mean(-1, keepdims=True)
    return (x - m) / jnp.sqrt(v + 1e-5) * g + b


def kernel(x, edge_index, edge_attr, params):
    N, D = x.shape
    E = edge_index.shape[1]
    H = 16
    DH = D // H
    src, dst = edge_index[0], edge_index[1]

    p = params
    BE = 3200 if E % 3200 == 0 else E

    w1_row = p['ee_W1'].T.reshape(1, -1)
    b1_row = p['ee_b1'].reshape(1, -1)
    wsum_row = p['ee_W2'].sum(0).reshape(1, -1)
    csum = p['ee_b2'].sum()
    r = jnp.maximum(edge_attr * w1_row + b1_row, 0.0)
    ew = jax.nn.sigmoid(jnp.sum(r * wsum_row, axis=1) + csum)

    h = x @ p['in_W'].T + p['in_b']

    src3s = src.reshape(E // _CHUNK, 1, _CHUNK)
    dst3s = dst.reshape(E // _CHUNK, 1, _CHUNK)
    src3g = src.reshape(E // (_NSTREAM * _CHUNK), _NSTREAM, _CHUNK)
    dst3g = dst.reshape(E // (_NSTREAM * _CHUNK), _NSTREAM, _CHUNK)
    ew1d = jnp.broadcast_to(ew[:, None], (E, 16)).reshape(E * 16)
    (c_parts,) = _sc_counts(dst, N)
    counts = c_parts[0, :N, 0] + c_parts[1, :N, 0]
    inv_cnt = 1.0 / jnp.maximum(counts, 1.0)
    for li, lp in enumerate(p['layers']):
        (p_parts,) = _sc_segsum(h, src3s, dst3s, ew1d)
        P = p_parts[0, :N, :D] + p_parts[1, :N, :D]
        S = (P @ lp['Wq'].T).reshape(N, H, DH)
        K = (h @ lp['Wk'].T).reshape(N, H, DH)
        V = (h @ lp['Wv'].T).reshape(N, H, DH)
        s = (K * S).sum(-1) * (1.0 / jnp.sqrt(jnp.float32(DH)))
        attn = (V * s[:, :, None] * inv_cnt[:, None, None]).reshape(N, D)
        attn = attn @ lp['Wo'].T + lp['bo']
        h1 = _layer_norm(h + attn, lp['ln1_g'], lp['ln1_b'])
        ff = jnp.maximum(h1 @ lp['ff_W1'].T + lp['ff_b1'], 0.0) @ lp['ff_W2'].T + lp['ff_b2']
        h = _layer_norm(h1 + ff, lp['ln2_g'], lp['ln2_b'])

    W1_all = jnp.concatenate([p['cls_W1'], p['pp_W1']], axis=0)
    wsrc = W1_all[:, :D].T
    wdst = W1_all[:, D:2 * D].T
    wr = p['ee_W2'].T @ W1_all[:, 2 * D:].T
    ball = (jnp.concatenate([p['cls_b1'], p['pp_b1']])
            + p['ee_b2'] @ W1_all[:, 2 * D:].T).reshape(1, -1)

    hs, hd = _sc_gather2(h, src3g, dst3g, E)
    logits, pars = _edge_head(
        hs, hd, r, wsrc, wdst, wr, ball,
        p['cls_W2'].T, p['cls_b2'].reshape(1, -1),
        p['pp_W2'].T, p['pp_b2'].reshape(1, -1),
        p['cls_W3'].T, p['cls_b3'].reshape(1, -1),
        p['pp_W3'].T, p['pp_b3'].reshape(1, -1), BE)
    return logits[:, 0], pars


# speedup vs baseline: 1.0382x; 1.0382x over previous
import functools
from typing import Any

import jax
import jax.numpy as jnp
from jax import lax
from jax.experimental import pallas as pl
from jax.experimental.pallas import tpu as pltpu
from jax.experimental.pallas import tpu_sc as plsc

_NCORES = 2
_NSUB = 16
_NW = _NCORES * _NSUB
_NLANES = 16
_CHUNK = 80
_NSTREAM = 5


def _sc_counts(dst, N):
    E = dst.shape[0]
    D = 128
    NP = ((N + 127) // 128) * 128
    RPS = NP // _NSUB
    EPW = E // _NW
    n_chunks = EPW // _CHUNK
    assert EPW % _CHUNK == 0

    mesh = plsc.VectorSubcoreMesh(core_axis_name="c", subcore_axis_name="s")
    out_type = [jax.ShapeDtypeStruct((_NCORES, NP, D), jnp.float32)]
    scratch = [
        pltpu.VMEM((_CHUNK,), jnp.int32),
        pltpu.VMEM((_CHUNK, D), jnp.float32),
        pltpu.VMEM_SHARED((NP, D), jnp.float32),
    ]

    def body(dst_hbm, c_out, dst_v, ones_v, acc):
        cid = lax.axis_index("c")
        sid = lax.axis_index("s")
        wid = sid * _NCORES + cid
        rsl = pl.ds(sid * RPS, RPS)
        zrow = jnp.zeros((_NLANES,), jnp.float32)

        def zbuf(e, _):
            for j in range(D // _NLANES):
                ones_v[e, pl.ds(j * _NLANES, _NLANES)] = zrow
            return 0
        lax.fori_loop(0, _CHUNK, zbuf, 0)
        nfull, rem = RPS // _CHUNK, RPS % _CHUNK
        for k in range(nfull):
            pltpu.sync_copy(ones_v, acc.at[pl.ds(sid * RPS + k * _CHUNK, _CHUNK)])
        if rem:
            pltpu.sync_copy(ones_v.at[pl.ds(0, rem)],
                            acc.at[pl.ds(sid * RPS + nfull * _CHUNK, rem)])
        one_row = jnp.where(lax.iota(jnp.int32, _NLANES) == 0, 1.0, 0.0)

        def initones(e, _):
            ones_v[e, pl.ds(0, _NLANES)] = one_row
            return 0
        lax.fori_loop(0, _CHUNK, initones, 0)
        plsc.subcore_barrier()

        def chunk(g, _):
            base = wid * EPW + g * _CHUNK
            pltpu.sync_copy(dst_hbm.at[pl.ds(base, _CHUNK)], dst_v)
            pltpu.sync_copy(ones_v, acc.at[dst_v], add=True)
            return 0
        lax.fori_loop(0, n_chunks, chunk, 0)
        plsc.subcore_barrier()
        pltpu.sync_copy(acc.at[rsl], c_out.at[cid, rsl])

    return pl.kernel(body, out_type=out_type, mesh=mesh, scratch_types=scratch)(dst)


def _sc_segsum(h, src3d, dst3d, ew):
    N, D = h.shape
    E = ew.shape[0] // _NLANES
    NP = ((N + 127) // 128) * 128
    RPS = NP // _NSUB
    EPW = E // _NW
    NS = 1
    IB = NS * _CHUNK
    n_iter = EPW // IB
    assert EPW % IB == 0 and n_iter % 2 == 1 and NP % _NSUB == 0

    mesh = plsc.VectorSubcoreMesh(core_axis_name="c", subcore_axis_name="s")
    out_type = [jax.ShapeDtypeStruct((_NCORES, NP, D), jnp.float32)]
    scratch = [
        pltpu.VMEM((1, NS, _CHUNK), jnp.int32),
        pltpu.VMEM((1, NS, _CHUNK), jnp.int32),
        pltpu.VMEM((1, NS, _CHUNK), jnp.int32),
        pltpu.VMEM((1, NS, _CHUNK), jnp.int32),
        pltpu.VMEM((IB * _NLANES,), jnp.float32),
        pltpu.VMEM((IB * _NLANES,), jnp.float32),
        pltpu.VMEM((IB, D), jnp.float32),
        pltpu.VMEM((IB, D), jnp.float32),
        pltpu.VMEM_SHARED((NP, D), jnp.float32),
        pltpu.SemaphoreType.DMA, pltpu.SemaphoreType.DMA,
        pltpu.SemaphoreType.DMA, pltpu.SemaphoreType.DMA,
    ]

    def body(h_hbm, src3_hbm, dst3_hbm, ew_hbm, p_out,
             sb0, sb1, db0, db1, eb0, eb1, rb0, rb1, acc,
             gs0, gs1, ss0, ss1):
        srcb, dstb, ewb, rows = (sb0, sb1), (db0, db1), (eb0, eb1), (rb0, rb1)
        gsem, ssem = (gs0, gs1), (ss0, ss1)
        cid = lax.axis_index("c")
        sid = lax.axis_index("s")
        wid = sid * _NCORES + cid
        rsl = pl.ds(sid * RPS, RPS)
        zrow = jnp.zeros((_NLANES,), jnp.float32)

        def zbuf(e, _):
            for j in range(D // _NLANES):
                rb0[e, pl.ds(j * _NLANES, _NLANES)] = zrow
            return 0
        lax.fori_loop(0, IB, zbuf, 0)
        nfull, rem = RPS // IB, RPS % IB
        for k in range(nfull):
            pltpu.sync_copy(rb0, acc.at[pl.ds(sid * RPS + k * IB, IB)])
        if rem:
            pltpu.sync_copy(rb0.at[pl.ds(0, rem)],
                            acc.at[pl.ds(sid * RPS + nfull * IB, rem)])
        plsc.subcore_barrier()

        def load_idx(g, b):
            blk = wid * n_iter + g
            pltpu.sync_copy(src3_hbm.at[pl.ds(blk, 1)], srcb[b])
            pltpu.sync_copy(dst3_hbm.at[pl.ds(blk, 1)], dstb[b])
            pltpu.sync_copy(
                ew_hbm.at[pl.ds((wid * EPW + g * IB) * _NLANES, IB * _NLANES)],
                ewb[b])

        def start_gathers(b):
            for j in range(NS):
                pltpu.async_copy(h_hbm.at[srcb[b].at[0, j]],
                                 rows[b].at[pl.ds(j * _CHUNK, _CHUNK)], gsem[b])

        def drain_gathers(b):
            for j in range(NS):
                pltpu.make_async_copy(
                    h_hbm.at[pl.ds(0, _CHUNK)],
                    rows[b].at[pl.ds(j * _CHUNK, _CHUNK)], gsem[b]).wait()

        def start_scatters(b):
            for j in range(NS):
                pltpu.async_copy(rows[b].at[pl.ds(j * _CHUNK, _CHUNK)],
                                 acc.at[dstb[b].at[0, j]], ssem[b], add=True)

        def drain_scatters(b):
            for j in range(NS):
                pltpu.make_async_copy(
                    rows[b].at[pl.ds(j * _CHUNK, _CHUNK)],
                    acc.at[pl.ds(0, _CHUNK)], ssem[b]).wait()

        def scale(b):
            def s1(e, _):
                w = ewb[b][pl.ds(e * _NLANES, _NLANES)]
                for j in range(D // _NLANES):
                    sl = pl.ds(j * _NLANES, _NLANES)
                    rows[b][e, sl] = rows[b][e, sl] * w
                return 0
            lax.fori_loop(0, IB, s1, 0)

        load_idx(0, 0)
        start_gathers(0)

        @pl.loop(0, n_iter - 1, step=2)
        def _(i0):
            for b in (0, 1):
                g = i0 + b
                nb = 1 - b

                @pl.when(g >= 1)
                def _():
                    drain_scatters(nb)
                load_idx(g + 1, nb)
                start_gathers(nb)
                drain_gathers(b)
                scale(b)
                start_scatters(b)

        drain_gathers(0)
        scale(0)
        start_scatters(0)
        drain_scatters(0)
        drain_scatters(1)
        plsc.subcore_barrier()
        pltpu.sync_copy(acc.at[rsl], p_out.at[cid, rsl])

    return pl.kernel(body, out_type=out_type, mesh=mesh, scratch_types=scratch)(
        h, src3d, dst3d, ew)


def _sc_gather2(h, src3d, dst3d, E):
    N, D = h.shape
    EPW = E // _NW
    IB = _NSTREAM * _CHUNK
    n_iter = EPW // IB
    assert EPW % IB == 0 and n_iter % 2 == 1

    mesh = plsc.VectorSubcoreMesh(core_axis_name="c", subcore_axis_name="s")
    out_type = [jax.ShapeDtypeStruct((E, D), jnp.float32),
                jax.ShapeDtypeStruct((E, D), jnp.float32)]
    scratch = [
        pltpu.VMEM((1, _NSTREAM, _CHUNK), jnp.int32),
        pltpu.VMEM((1, _NSTREAM, _CHUNK), jnp.int32),
        pltpu.VMEM((IB, D), jnp.float32),
        pltpu.VMEM((IB, D), jnp.float32),
        pltpu.SemaphoreType.DMA, pltpu.SemaphoreType.DMA,
        pltpu.SemaphoreType.DMA, pltpu.SemaphoreType.DMA,
    ]

    def body(h_hbm, src3_hbm, dst3_hbm, hs_out, hd_out,
             ib0, ib1, rb0, rb1, gs0, gs1, ws0, ws1):
        idxb, rows = (ib0, ib1), (rb0, rb1)
        gsem, wsem = (gs0, gs1), (ws0, ws1)
        cid = lax.axis_index("c")
        sid = lax.axis_index("s")
        wid = sid * _NCORES + cid

        for idx3_hbm, out_hbm in ((src3_hbm, hs_out), (dst3_hbm, hd_out)):
            def load_idx(g, b):
                pltpu.sync_copy(idx3_hbm.at[pl.ds(wid * n_iter + g, 1)], idxb[b])

            def start_gathers(b):
                for j in range(_NSTREAM):
                    pltpu.async_copy(h_hbm.at[idxb[b].at[0, j]],
                                     rows[b].at[pl.ds(j * _CHUNK, _CHUNK)],
                                     gsem[b])

            def drain_gathers(b):
                for j in range(_NSTREAM):
                    pltpu.make_async_copy(
                        h_hbm.at[pl.ds(0, _CHUNK)],
                        rows[b].at[pl.ds(j * _CHUNK, _CHUNK)], gsem[b]).wait()

            def start_write(g, b):
                pltpu.async_copy(rows[b],
                                 out_hbm.at[pl.ds(wid * EPW + g * IB, IB)],
                                 wsem[b])

            def drain_write(g, b):
                pltpu.make_async_copy(
                    rows[b], out_hbm.at[pl.ds(wid * EPW + g * IB, IB)],
                    wsem[b]).wait()

            load_idx(0, 0)
            start_gathers(0)

            @pl.loop(0, n_iter - 1, step=2)
            def _(i0):
                for b in (0, 1):
                    g = i0 + b
                    nb = 1 - b

                    @pl.when(g >= 1)
                    def _():
                        drain_write(g - 1, nb)
                    load_idx(g + 1, nb)
                    start_gathers(nb)
                    drain_gathers(b)
                    start_write(g, b)

            drain_gathers(0)
            start_write(n_iter - 1, 0)
            drain_write(n_iter - 1, 0)
            drain_write(n_iter - 2, 1)

    return pl.kernel(body, out_type=out_type, mesh=mesh, scratch_types=scratch)(
        h, src3d, dst3d)


def _edge_head_body(hs_ref, hd_ref, a_ref, w1_ref, b1_ref,
                    wsrc_ref, wdst_ref, wr_ref, ball_ref,
                    w2c_ref, b2c_ref, w2p_ref, b2p_ref, w3c_ref, b3c_ref,
                    w3p_ref, b3p_ref, logit_ref, par_ref):
    hs = hs_ref[...]
    hd = hd_ref[...]
    a = a_ref[...]
    r = jnp.maximum(a * w1_ref[...] + b1_ref[...], 0.0)
    u = (jnp.dot(hs, wsrc_ref[...], preferred_element_type=jnp.float32)
         + jnp.dot(hd, wdst_ref[...], preferred_element_type=jnp.float32)
         + jnp.dot(r, wr_ref[...], preferred_element_type=jnp.float32)
         + ball_ref[...])
    u = jnp.maximum(u, 0.0)
    c = u[:, :128]
    q = u[:, 128:]
    c2 = jnp.maximum(jnp.dot(c, w2c_ref[...], preferred_element_type=jnp.float32)
                     + b2c_ref[...], 0.0)
    q2 = jnp.maximum(jnp.dot(q, w2p_ref[...], preferred_element_type=jnp.float32)
                     + b2p_ref[...], 0.0)
    lg = jnp.dot(c2, w3c_ref[...], preferred_element_type=jnp.float32) + b3c_ref[...]
    logit_ref[...] = lg
    pp = jnp.dot(q2, w3p_ref[...], preferred_element_type=jnp.float32) + b3p_ref[...]
    par_ref[...] = jax.nn.softplus(pp) + 1e-6


def _edge_head(hs, hd, a, w1_row, b1_row, wsrc, wdst, wr, ball, w2c, b2c,
               w2p, b2p, w3c, b3c, w3p, b3p, BE):
    E = hs.shape[0]
    grid = (E // BE,)
    wb = lambda shape: pl.BlockSpec(shape, lambda i: tuple(0 for _ in shape))
    return pl.pallas_call(
        _edge_head_body,
        grid=grid,
        in_specs=[
            pl.BlockSpec((BE, 128), lambda i: (i, 0)),
            pl.BlockSpec((BE, 128), lambda i: (i, 0)),
            pl.BlockSpec((BE, 1), lambda i: (i, 0)),
            wb((1, 32)), wb((1, 32)),
            wb((128, 256)), wb((128, 256)), wb((32, 256)), wb((1, 256)),
            wb((128, 64)), wb((1, 64)), wb((128, 64)), wb((1, 64)),
            wb((64, 1)), wb((1, 1)), wb((64, 4)), wb((1, 4)),
        ],
        out_specs=[
            pl.BlockSpec((BE, 1), lambda i: (i, 0)),
            pl.BlockSpec((BE, 4), lambda i: (i, 0)),
        ],
        out_shape=[
            jax.ShapeDtypeStruct((E, 1), jnp.float32),
            jax.ShapeDtypeStruct((E, 4), jnp.float32),
        ],
    )(hs, hd, a, w1_row, b1_row, wsrc, wdst, wr, ball,
      w2c, b2c, w2p, b2p, w3c, b3c, w3p, b3p)


def _layer_norm(x, g, b):
    m = x.mean(-1, keepdims=True)
    v = ((x - m) ** 2).mean(-1, keepdims=True)
    return (x - m) / jnp.sqrt(v + 1e-5) * g + b


def kernel(x, edge_index, edge_attr, params):
    N, D = x.shape
    E = edge_index.shape[1]
    H = 16
    DH = D // H
    src, dst = edge_index[0], edge_index[1]

    p = params
    BE = 3200 if E % 3200 == 0 else E

    w1_row = p['ee_W1'].T.reshape(1, -1)
    b1_row = p['ee_b1'].reshape(1, -1)
    wsum_row = p['ee_W2'].sum(0).reshape(1, -1)
    csum = p['ee_b2'].sum()
    rtmp = jnp.maximum(edge_attr * w1_row + b1_row, 0.0)
    ew = jax.nn.sigmoid(jnp.sum(rtmp * wsum_row, axis=1) + csum)

    h = x @ p['in_W'].T + p['in_b']

    src3s = src.reshape(E // _CHUNK, 1, _CHUNK)
    dst3s = dst.reshape(E // _CHUNK, 1, _CHUNK)
    src3g = src.reshape(E // (_NSTREAM * _CHUNK), _NSTREAM, _CHUNK)
    dst3g = dst.reshape(E // (_NSTREAM * _CHUNK), _NSTREAM, _CHUNK)
    ew1d = jnp.broadcast_to(ew[:, None], (E, 16)).reshape(E * 16)
    (c_parts,) = _sc_counts(dst, N)
    counts = c_parts[0, :N, 0] + c_parts[1, :N, 0]
    inv_cnt = 1.0 / jnp.maximum(counts, 1.0)
    for li, lp in enumerate(p['layers']):
        (p_parts,) = _sc_segsum(h, src3s, dst3s, ew1d)
        P = p_parts[0, :N, :D] + p_parts[1, :N, :D]
        S = (P @ lp['Wq'].T).reshape(N, H, DH)
        K = (h @ lp['Wk'].T).reshape(N, H, DH)
        V = (h @ lp['Wv'].T).reshape(N, H, DH)
        s = (K * S).sum(-1) * (1.0 / jnp.sqrt(jnp.float32(DH)))
        attn = (V * s[:, :, None] * inv_cnt[:, None, None]).reshape(N, D)
        attn = attn @ lp['Wo'].T + lp['bo']
        h1 = _layer_norm(h + attn, lp['ln1_g'], lp['ln1_b'])
        ff = jnp.maximum(h1 @ lp['ff_W1'].T + lp['ff_b1'], 0.0) @ lp['ff_W2'].T + lp['ff_b2']
        h = _layer_norm(h1 + ff, lp['ln2_g'], lp['ln2_b'])

    W1_all = jnp.concatenate([p['cls_W1'], p['pp_W1']], axis=0)
    wsrc = W1_all[:, :D].T
    wdst = W1_all[:, D:2 * D].T
    wr = p['ee_W2'].T @ W1_all[:, 2 * D:].T
    ball = (jnp.concatenate([p['cls_b1'], p['pp_b1']])
            + p['ee_b2'] @ W1_all[:, 2 * D:].T).reshape(1, -1)

    hs, hd = _sc_gather2(h, src3g, dst3g, E)
    logits, pars = _edge_head(
        hs, hd, edge_attr, w1_row, b1_row, wsrc, wdst, wr, ball,
        p['cls_W2'].T, p['cls_b2'].reshape(1, -1),
        p['pp_W2'].T, p['pp_b2'].reshape(1, -1),
        p['cls_W3'].T, p['cls_b3'].reshape(1, -1),
        p['pp_W3'].T, p['pp_b3'].reshape(1, -1), BE)
    return logits[:, 0], pars
